# baseline (speedup 1.0000x reference)
import functools

import numpy as np

import jax
import jax.numpy as jnp
from jax import lax
from jax.experimental import pallas as pl
from jax.experimental.pallas import tpu as pltpu
from jax.experimental.pallas import tpu_sc as plsc

VOCAB = 1000000
D = 64
B = 4096
L = 200
NC = 2
NS = 16
NW = NC * NS
PER_W = B // NW
L_LO = 104
L_HI = L - L_LO
LANES = 16
DW = 32
NCH = D // LANES

TBLK = 32768
QUART = TBLK // 4
TILE = 512
NBLK = -(-VOCAB // TBLK)
ROWS = NBLK * QUART
VOCAB_PAD = 4 * ROWS

_PERM = np.concatenate(
    [np.arange(0, 16), np.arange(32, 48),
     np.arange(16, 32), np.arange(48, 64)])


def _repack_tc(emb_t):
    def kern(in_ref, o_ref):
        for t in range(QUART // TILE):
            slabs = [
                in_ref[:, pl.ds(QUART * q + TILE * t, TILE)]
                for q in range(4)
            ]
            tt = jnp.transpose(jnp.concatenate(slabs, axis=0))
            u = lax.bitcast_convert_type(tt, jnp.uint32)
            r = u + jnp.uint32(0x7FFF) + ((u >> 16) & jnp.uint32(1))
            a = jnp.concatenate(
                [r[:, 64 * q : 64 * q + 32] for q in range(4)], axis=1)
            b = jnp.concatenate(
                [r[:, 64 * q + 32 : 64 * q + 64] for q in range(4)], axis=1)
            packed = (a >> 16) | (b & jnp.uint32(0xFFFF0000))
            o_ref[pl.ds(TILE * t, TILE), :] = lax.bitcast_convert_type(
                packed, jnp.float32)

    return pl.pallas_call(
        kern,
        grid=(NBLK,),
        in_specs=[pl.BlockSpec((D, TBLK), lambda i: (0, i))],
        out_specs=pl.BlockSpec((QUART, 4 * DW), lambda i: (i, 0)),
        out_shape=jax.ShapeDtypeStruct((ROWS, 4 * DW), jnp.float32),
    )(emb_t)


def _remap_idx(x):
    v = x.astype(jnp.int32)
    v = ((v >> 15) << 15) + ((v & (QUART - 1)) << 2) + ((v >> 13) & 3)
    return v.reshape(B * L)


def _pooled_sc(x_flat, table):
    mesh = plsc.VectorSubcoreMesh(core_axis_name="c", subcore_axis_name="s")

    @functools.partial(
        pl.kernel,
        out_type=jax.ShapeDtypeStruct((B, D), jnp.float32),
        mesh=mesh,
        scratch_types=[
            pltpu.VMEM((PER_W * L,), jnp.int32),
            pltpu.VMEM((L, DW), jnp.float32),
            pltpu.VMEM((L, DW), jnp.float32),
            pltpu.VMEM((L, DW), jnp.float32),
            pltpu.VMEM((L, DW), jnp.float32),
            pltpu.VMEM((PER_W, D), jnp.float32),
            pltpu.SemaphoreType.DMA,
            pltpu.SemaphoreType.DMA,
            pltpu.SemaphoreType.DMA,
            pltpu.SemaphoreType.DMA,
        ],
        compiler_params=pltpu.CompilerParams(
            use_tc_tiling_on_sc=False, needs_layout_passes=False),
    )
    def kern(x_hbm, tab_hbm, out_hbm, idx_v, rows0, rows1, rows2, rows3,
             out_v, sem0, sem1, sem2, sem3):
        cid = lax.axis_index("c")
        sid = lax.axis_index("s")
        wid = sid * NC + cid
        base = pl.multiple_of(wid * PER_W, PER_W)

        pltpu.sync_copy(
            x_hbm.at[pl.ds(pl.multiple_of(wid * (PER_W * L), 8), PER_W * L)],
            idx_v)

        scale = jnp.float32(1.0 / L)
        himask = jnp.full((LANES,), -65536, jnp.int32)

        def issue(i, buf, sem):
            off = pl.multiple_of(i * L, 8)
            pltpu.async_copy(
                tab_hbm.at[idx_v.at[pl.ds(off, L_LO)]],
                buf.at[pl.ds(0, L_LO)], sem)
            pltpu.async_copy(
                tab_hbm.at[idx_v.at[pl.ds(off + L_LO, L_HI)]],
                buf.at[pl.ds(L_LO, L_HI)], sem)

        def wait(buf, sem):
            pltpu.make_async_copy(tab_hbm.at[pl.ds(0, L)], buf, sem).wait()

        def accum(buf, i):
            zeros = (jnp.zeros((LANES,), jnp.float32),) * NCH

            @pl.loop(0, L, init_carry=zeros, unroll=8)
            def red(r, acc):
                new = []
                for c in range(2):
                    w = buf[r, pl.ds(c * LANES, LANES)]
                    u = plsc.bitcast(w, jnp.int32)
                    lo = plsc.bitcast(u << 16, jnp.float32)
                    hi = plsc.bitcast(u & himask, jnp.float32)
                    new.append(acc[2 * c] + lo)
                    new.append(acc[2 * c + 1] + hi)
                return tuple(new)

            for j in range(NCH):
                out_v[i, pl.ds(j * LANES, LANES)] = red[j] * scale

        bufs = (rows0, rows1, rows2, rows3)
        sems = (sem0, sem1, sem2, sem3)
        NBUF = 4

        for k in range(NBUF - 1):
            issue(k, bufs[k], sems[k])

        @pl.loop(0, PER_W, step=NBUF)
        def elem(i):
            for k in range(NBUF):
                nxt = i + k + NBUF - 1

                @pl.when(nxt < PER_W)
                def _():
                    issue(nxt, bufs[(k + NBUF - 1) % NBUF],
                          sems[(k + NBUF - 1) % NBUF])

                wait(bufs[k], sems[k])
                accum(bufs[k], i + k)

        pltpu.sync_copy(out_v, out_hbm.at[pl.ds(base, PER_W)])

    return kern(x_flat, table)


def _linear_tc(pooled, w, b2):
    def kern(p_ref, w_ref, b_ref, o_ref):
        o_ref[...] = lax.dot_general(
            p_ref[...], w_ref[...], (((1,), (1,)), ((), ())),
            preferred_element_type=jnp.float32) + b_ref[...]

    return pl.pallas_call(
        kern,
        out_shape=jax.ShapeDtypeStruct((B, 2), jnp.float32),
    )(pooled, w, b2)


def kernel(x, embedding, fc_w, fc_b):
    table = _repack_tc(embedding.T).reshape(VOCAB_PAD, DW)
    pooled = _pooled_sc(_remap_idx(x), table)
    return _linear_tc(pooled, fc_w[:, _PERM], fc_b.reshape(1, 2))


# speedup vs baseline: 1.4438x; 1.4438x over previous
import functools

import numpy as np

import jax
import jax.numpy as jnp
from jax import lax
from jax.experimental import pallas as pl
from jax.experimental.pallas import tpu as pltpu
from jax.experimental.pallas import tpu_sc as plsc

VOCAB = 1000000
D = 64
B = 4096
L = 200
NC = 2
NS = 16
NW = NC * NS
PER_W = B // NW
L_LO = 104
L_HI = L - L_LO
LANES = 16
DW = 32
NCH = D // LANES

TBLK = 32768
QUART = TBLK // 4
TILE = 512
NBLK = -(-VOCAB // TBLK)
ROWS = NBLK * QUART
VOCAB_PAD = 4 * ROWS

_PERM = np.concatenate(
    [np.arange(0, 16), np.arange(32, 48),
     np.arange(16, 32), np.arange(48, 64)])


def _repack_tc(emb_t):
    def kern(in_ref, o_ref):
        for t in range(QUART // TILE):
            packed_slabs = []
            for q in range(4):
                s = in_ref[:, pl.ds(QUART * q + TILE * t, TILE)]
                u = lax.bitcast_convert_type(s, jnp.uint32)
                r = u + jnp.uint32(0x7FFF) + ((u >> 16) & jnp.uint32(1))
                packed_slabs.append(
                    (r[0:32, :] >> 16) | (r[32:64, :] & jnp.uint32(0xFFFF0000)))
            blk = jnp.concatenate(packed_slabs, axis=0)
            o_ref[pl.ds(TILE * t, TILE), :] = lax.bitcast_convert_type(
                jnp.transpose(blk), jnp.float32)

    return pl.pallas_call(
        kern,
        grid=(NBLK,),
        in_specs=[pl.BlockSpec((D, TBLK), lambda i: (0, i))],
        out_specs=pl.BlockSpec((QUART, 4 * DW), lambda i: (i, 0)),
        out_shape=jax.ShapeDtypeStruct((ROWS, 4 * DW), jnp.float32),
    )(emb_t)


def _remap_idx(x):
    v = x.astype(jnp.int32)
    v = ((v >> 15) << 15) + ((v & (QUART - 1)) << 2) + ((v >> 13) & 3)
    return v.reshape(B * L)


def _pooled_sc(x_flat, table):
    mesh = plsc.VectorSubcoreMesh(core_axis_name="c", subcore_axis_name="s")

    @functools.partial(
        pl.kernel,
        out_type=jax.ShapeDtypeStruct((B, D), jnp.float32),
        mesh=mesh,
        scratch_types=[
            pltpu.VMEM((PER_W * L,), jnp.int32),
            pltpu.VMEM((L, DW), jnp.float32),
            pltpu.VMEM((L, DW), jnp.float32),
            pltpu.VMEM((L, DW), jnp.float32),
            pltpu.VMEM((L, DW), jnp.float32),
            pltpu.VMEM((PER_W, D), jnp.float32),
            pltpu.SemaphoreType.DMA,
            pltpu.SemaphoreType.DMA,
            pltpu.SemaphoreType.DMA,
            pltpu.SemaphoreType.DMA,
        ],
        compiler_params=pltpu.CompilerParams(
            use_tc_tiling_on_sc=False, needs_layout_passes=False),
    )
    def kern(x_hbm, tab_hbm, out_hbm, idx_v, rows0, rows1, rows2, rows3,
             out_v, sem0, sem1, sem2, sem3):
        cid = lax.axis_index("c")
        sid = lax.axis_index("s")
        wid = sid * NC + cid
        base = pl.multiple_of(wid * PER_W, PER_W)

        pltpu.sync_copy(
            x_hbm.at[pl.ds(pl.multiple_of(wid * (PER_W * L), 8), PER_W * L)],
            idx_v)

        scale = jnp.float32(1.0 / L)
        himask = jnp.full((LANES,), -65536, jnp.int32)

        def issue(i, buf, sem):
            off = pl.multiple_of(i * L, 8)
            pltpu.async_copy(
                tab_hbm.at[idx_v.at[pl.ds(off, L_LO)]],
                buf.at[pl.ds(0, L_LO)], sem)
            pltpu.async_copy(
                tab_hbm.at[idx_v.at[pl.ds(off + L_LO, L_HI)]],
                buf.at[pl.ds(L_LO, L_HI)], sem)

        def wait(buf, sem):
            pltpu.make_async_copy(tab_hbm.at[pl.ds(0, L)], buf, sem).wait()

        def accum(buf, i):
            zeros = (jnp.zeros((LANES,), jnp.float32),) * NCH

            @pl.loop(0, L, init_carry=zeros, unroll=8)
            def red(r, acc):
                new = []
                for c in range(2):
                    w = buf[r, pl.ds(c * LANES, LANES)]
                    u = plsc.bitcast(w, jnp.int32)
                    lo = plsc.bitcast(u << 16, jnp.float32)
                    hi = plsc.bitcast(u & himask, jnp.float32)
                    new.append(acc[2 * c] + lo)
                    new.append(acc[2 * c + 1] + hi)
                return tuple(new)

            for j in range(NCH):
                out_v[i, pl.ds(j * LANES, LANES)] = red[j] * scale

        bufs = (rows0, rows1, rows2, rows3)
        sems = (sem0, sem1, sem2, sem3)
        NBUF = 4

        for k in range(NBUF - 1):
            issue(k, bufs[k], sems[k])

        @pl.loop(0, PER_W, step=NBUF)
        def elem(i):
            for k in range(NBUF):
                nxt = i + k + NBUF - 1

                @pl.when(nxt < PER_W)
                def _():
                    issue(nxt, bufs[(k + NBUF - 1) % NBUF],
                          sems[(k + NBUF - 1) % NBUF])

                wait(bufs[k], sems[k])
                accum(bufs[k], i + k)

        pltpu.sync_copy(out_v, out_hbm.at[pl.ds(base, PER_W)])

    return kern(x_flat, table)


def _linear_tc(pooled, w, b2):
    def kern(p_ref, w_ref, b_ref, o_ref):
        o_ref[...] = lax.dot_general(
            p_ref[...], w_ref[...], (((1,), (1,)), ((), ())),
            preferred_element_type=jnp.float32) + b_ref[...]

    return pl.pallas_call(
        kern,
        out_shape=jax.ShapeDtypeStruct((B, 2), jnp.float32),
    )(pooled, w, b2)


def kernel(x, embedding, fc_w, fc_b):
    table = _repack_tc(embedding.T).reshape(VOCAB_PAD, DW)
    pooled = _pooled_sc(_remap_idx(x), table)
    return _linear_tc(pooled, fc_w[:, _PERM], fc_b.reshape(1, 2))


# speedup vs baseline: 1.4526x; 1.0061x over previous
import functools

import numpy as np

import jax
import jax.numpy as jnp
from jax import lax
from jax.experimental import pallas as pl
from jax.experimental.pallas import tpu as pltpu
from jax.experimental.pallas import tpu_sc as plsc

VOCAB = 1000000
D = 64
B = 4096
L = 200
NC = 2
NS = 16
NW = NC * NS
PER_W = B // NW
L_LO = 104
L_HI = L - L_LO
LANES = 16
DW = 32
NCH = D // LANES

TBLK = 65536
QUART = TBLK // 4
TILE = 512
NBLK = -(-VOCAB // TBLK)
ROWS = NBLK * QUART
VOCAB_PAD = 4 * ROWS
_SB = TBLK.bit_length() - 1
_SQ = _SB - 2

_PERM = np.concatenate(
    [np.arange(0, 16), np.arange(32, 48),
     np.arange(16, 32), np.arange(48, 64)])


def _repack_tc(emb_t):
    def kern(in_ref, o_ref):
        for t in range(QUART // TILE):
            packed_slabs = []
            for q in range(4):
                s = in_ref[:, pl.ds(QUART * q + TILE * t, TILE)]
                u = lax.bitcast_convert_type(s, jnp.uint32)
                r = u + jnp.uint32(0x7FFF) + ((u >> 16) & jnp.uint32(1))
                packed_slabs.append(
                    (r[0:32, :] >> 16) | (r[32:64, :] & jnp.uint32(0xFFFF0000)))
            blk = jnp.concatenate(packed_slabs, axis=0)
            o_ref[pl.ds(TILE * t, TILE), :] = lax.bitcast_convert_type(
                jnp.transpose(blk), jnp.float32)

    return pl.pallas_call(
        kern,
        grid=(NBLK,),
        in_specs=[pl.BlockSpec((D, TBLK), lambda i: (0, i))],
        out_specs=pl.BlockSpec((QUART, 4 * DW), lambda i: (i, 0)),
        out_shape=jax.ShapeDtypeStruct((ROWS, 4 * DW), jnp.float32),
    )(emb_t)


def _remap_idx(x):
    v = x.astype(jnp.int32)
    v = ((v >> _SB) << _SB) + ((v & (QUART - 1)) << 2) + ((v >> _SQ) & 3)
    return v.reshape(B * L)


def _pooled_sc(x_flat, table):
    mesh = plsc.VectorSubcoreMesh(core_axis_name="c", subcore_axis_name="s")

    @functools.partial(
        pl.kernel,
        out_type=jax.ShapeDtypeStruct((B, D), jnp.float32),
        mesh=mesh,
        scratch_types=[
            pltpu.VMEM((PER_W * L,), jnp.int32),
            pltpu.VMEM((L, DW), jnp.float32),
            pltpu.VMEM((L, DW), jnp.float32),
            pltpu.VMEM((L, DW), jnp.float32),
            pltpu.VMEM((L, DW), jnp.float32),
            pltpu.VMEM((PER_W, D), jnp.float32),
            pltpu.SemaphoreType.DMA,
            pltpu.SemaphoreType.DMA,
            pltpu.SemaphoreType.DMA,
            pltpu.SemaphoreType.DMA,
        ],
        compiler_params=pltpu.CompilerParams(
            use_tc_tiling_on_sc=False, needs_layout_passes=False),
    )
    def kern(x_hbm, tab_hbm, out_hbm, idx_v, rows0, rows1, rows2, rows3,
             out_v, sem0, sem1, sem2, sem3):
        cid = lax.axis_index("c")
        sid = lax.axis_index("s")
        wid = sid * NC + cid
        base = pl.multiple_of(wid * PER_W, PER_W)

        pltpu.sync_copy(
            x_hbm.at[pl.ds(pl.multiple_of(wid * (PER_W * L), 8), PER_W * L)],
            idx_v)

        scale = jnp.float32(1.0 / L)
        himask = jnp.full((LANES,), -65536, jnp.int32)

        def issue(i, buf, sem):
            off = pl.multiple_of(i * L, 8)
            pltpu.async_copy(
                tab_hbm.at[idx_v.at[pl.ds(off, L_LO)]],
                buf.at[pl.ds(0, L_LO)], sem)
            pltpu.async_copy(
                tab_hbm.at[idx_v.at[pl.ds(off + L_LO, L_HI)]],
                buf.at[pl.ds(L_LO, L_HI)], sem)

        def wait(buf, sem):
            pltpu.make_async_copy(tab_hbm.at[pl.ds(0, L)], buf, sem).wait()

        def accum(buf, i):
            zeros = (jnp.zeros((LANES,), jnp.float32),) * NCH

            @pl.loop(0, L, init_carry=zeros, unroll=8)
            def red(r, acc):
                new = []
                for c in range(2):
                    w = buf[r, pl.ds(c * LANES, LANES)]
                    u = plsc.bitcast(w, jnp.int32)
                    lo = plsc.bitcast(u << 16, jnp.float32)
                    hi = plsc.bitcast(u & himask, jnp.float32)
                    new.append(acc[2 * c] + lo)
                    new.append(acc[2 * c + 1] + hi)
                return tuple(new)

            for j in range(NCH):
                out_v[i, pl.ds(j * LANES, LANES)] = red[j] * scale

        bufs = (rows0, rows1, rows2, rows3)
        sems = (sem0, sem1, sem2, sem3)
        NBUF = 4

        for k in range(NBUF - 1):
            issue(k, bufs[k], sems[k])

        @pl.loop(0, PER_W, step=NBUF)
        def elem(i):
            for k in range(NBUF):
                nxt = i + k + NBUF - 1

                @pl.when(nxt < PER_W)
                def _():
                    issue(nxt, bufs[(k + NBUF - 1) % NBUF],
                          sems[(k + NBUF - 1) % NBUF])

                wait(bufs[k], sems[k])
                accum(bufs[k], i + k)

        pltpu.sync_copy(out_v, out_hbm.at[pl.ds(base, PER_W)])

    return kern(x_flat, table)


def _linear_tc(pooled, w, b2):
    def kern(p_ref, w_ref, b_ref, o_ref):
        o_ref[...] = lax.dot_general(
            p_ref[...], w_ref[...], (((1,), (1,)), ((), ())),
            preferred_element_type=jnp.float32) + b_ref[...]

    return pl.pallas_call(
        kern,
        out_shape=jax.ShapeDtypeStruct((B, 2), jnp.float32),
    )(pooled, w, b2)


def kernel(x, embedding, fc_w, fc_b):
    table = _repack_tc(embedding.T).reshape(VOCAB_PAD, DW)
    pooled = _pooled_sc(_remap_idx(x), table)
    return _linear_tc(pooled, fc_w[:, _PERM], fc_b.reshape(1, 2))


# speedup vs baseline: 1.4658x; 1.0091x over previous
import functools

import numpy as np

import jax
import jax.numpy as jnp
from jax import lax
from jax.experimental import pallas as pl
from jax.experimental.pallas import tpu as pltpu
from jax.experimental.pallas import tpu_sc as plsc

VOCAB = 1000000
D = 64
B = 4096
L = 200
NC = 2
NS = 16
NW = NC * NS
PER_W = B // NW
L_LO = 104
L_HI = L - L_LO
LANES = 16
DW = 32
NCH = D // LANES

TBLK = 65536
QUART = TBLK // 4
TILE = 512
NBLK = -(-VOCAB // TBLK)
ROWS = NBLK * QUART
VOCAB_PAD = 4 * ROWS
_SB = TBLK.bit_length() - 1
_SQ = _SB - 2

_PERM = np.concatenate(
    [np.arange(0, 16), np.arange(32, 48),
     np.arange(16, 32), np.arange(48, 64)])


def _repack_tc(emb_t):
    def kern(in_ref, o_ref):
        for t in range(QUART // TILE):
            packed_slabs = []
            for q in range(4):
                s = in_ref[:, pl.ds(QUART * q + TILE * t, TILE)]
                u = lax.bitcast_convert_type(s, jnp.uint32)
                r = u + jnp.uint32(0x7FFF) + ((u >> 16) & jnp.uint32(1))
                packed_slabs.append(
                    (r[0:32, :] >> 16) | (r[32:64, :] & jnp.uint32(0xFFFF0000)))
            blk = jnp.concatenate(packed_slabs, axis=0)
            o_ref[pl.ds(TILE * t, TILE), :] = lax.bitcast_convert_type(
                jnp.transpose(blk), jnp.float32)

    return pl.pallas_call(
        kern,
        grid=(NBLK,),
        in_specs=[pl.BlockSpec((D, TBLK), lambda i: (0, i))],
        out_specs=pl.BlockSpec((QUART, 4 * DW), lambda i: (i, 0)),
        out_shape=jax.ShapeDtypeStruct((ROWS, 4 * DW), jnp.float32),
    )(emb_t)


def _remap_idx(x):
    v = x.astype(jnp.int32)
    v = ((v >> _SB) << _SB) + ((v & (QUART - 1)) << 2) + ((v >> _SQ) & 3)
    return v.reshape(B * L)


def _pooled_sc(x_flat, table):
    mesh = plsc.VectorSubcoreMesh(core_axis_name="c", subcore_axis_name="s")

    @functools.partial(
        pl.kernel,
        out_type=jax.ShapeDtypeStruct((B, D), jnp.float32),
        mesh=mesh,
        scratch_types=[
            pltpu.VMEM((PER_W * L,), jnp.int32),
            pltpu.VMEM((L, DW), jnp.float32),
            pltpu.VMEM((L, DW), jnp.float32),
            pltpu.VMEM((L, DW), jnp.float32),
            pltpu.VMEM((L, DW), jnp.float32),
            pltpu.VMEM((L, DW), jnp.float32),
            pltpu.VMEM((L, DW), jnp.float32),
            pltpu.VMEM((L, DW), jnp.float32),
            pltpu.VMEM((L, DW), jnp.float32),
            pltpu.VMEM((PER_W, D), jnp.float32),
            pltpu.SemaphoreType.DMA,
            pltpu.SemaphoreType.DMA,
            pltpu.SemaphoreType.DMA,
            pltpu.SemaphoreType.DMA,
            pltpu.SemaphoreType.DMA,
            pltpu.SemaphoreType.DMA,
            pltpu.SemaphoreType.DMA,
            pltpu.SemaphoreType.DMA,
        ],
        compiler_params=pltpu.CompilerParams(
            use_tc_tiling_on_sc=False, needs_layout_passes=False),
    )
    def kern(x_hbm, tab_hbm, out_hbm, idx_v, rows0, rows1, rows2, rows3,
             rows4, rows5, rows6, rows7, out_v,
             sem0, sem1, sem2, sem3, sem4, sem5, sem6, sem7):
        cid = lax.axis_index("c")
        sid = lax.axis_index("s")
        wid = sid * NC + cid
        base = pl.multiple_of(wid * PER_W, PER_W)

        pltpu.sync_copy(
            x_hbm.at[pl.ds(pl.multiple_of(wid * (PER_W * L), 8), PER_W * L)],
            idx_v)

        scale = jnp.float32(1.0 / L)
        himask = jnp.full((LANES,), -65536, jnp.int32)

        def issue(i, buf, sem):
            off = pl.multiple_of(i * L, 8)
            pltpu.async_copy(
                tab_hbm.at[idx_v.at[pl.ds(off, L_LO)]],
                buf.at[pl.ds(0, L_LO)], sem)
            pltpu.async_copy(
                tab_hbm.at[idx_v.at[pl.ds(off + L_LO, L_HI)]],
                buf.at[pl.ds(L_LO, L_HI)], sem)

        def wait(buf, sem):
            pltpu.make_async_copy(tab_hbm.at[pl.ds(0, L)], buf, sem).wait()

        def accum(buf, i):
            zeros = (jnp.zeros((LANES,), jnp.float32),) * NCH

            @pl.loop(0, L, init_carry=zeros, unroll=8)
            def red(r, acc):
                new = []
                for c in range(2):
                    w = buf[r, pl.ds(c * LANES, LANES)]
                    u = plsc.bitcast(w, jnp.int32)
                    lo = plsc.bitcast(u << 16, jnp.float32)
                    hi = plsc.bitcast(u & himask, jnp.float32)
                    new.append(acc[2 * c] + lo)
                    new.append(acc[2 * c + 1] + hi)
                return tuple(new)

            for j in range(NCH):
                out_v[i, pl.ds(j * LANES, LANES)] = red[j] * scale

        bufs = (rows0, rows1, rows2, rows3, rows4, rows5, rows6, rows7)
        sems = (sem0, sem1, sem2, sem3, sem4, sem5, sem6, sem7)
        NBUF = 8

        for k in range(NBUF - 1):
            issue(k, bufs[k], sems[k])

        @pl.loop(0, PER_W, step=NBUF)
        def elem(i):
            for k in range(NBUF):
                nxt = i + k + NBUF - 1

                @pl.when(nxt < PER_W)
                def _():
                    issue(nxt, bufs[(k + NBUF - 1) % NBUF],
                          sems[(k + NBUF - 1) % NBUF])

                wait(bufs[k], sems[k])
                accum(bufs[k], i + k)

        pltpu.sync_copy(out_v, out_hbm.at[pl.ds(base, PER_W)])

    return kern(x_flat, table)


def _linear_tc(pooled, w, b2):
    def kern(p_ref, w_ref, b_ref, o_ref):
        o_ref[...] = lax.dot_general(
            p_ref[...], w_ref[...], (((1,), (1,)), ((), ())),
            preferred_element_type=jnp.float32) + b_ref[...]

    return pl.pallas_call(
        kern,
        out_shape=jax.ShapeDtypeStruct((B, 2), jnp.float32),
    )(pooled, w, b2)


def kernel(x, embedding, fc_w, fc_b):
    table = _repack_tc(embedding.T).reshape(VOCAB_PAD, DW)
    pooled = _pooled_sc(_remap_idx(x), table)
    return _linear_tc(pooled, fc_w[:, _PERM], fc_b.reshape(1, 2))
